# baseline (speedup 1.0000x reference)
import jax
import jax.numpy as jnp
from jax.experimental import pallas as pl

_BM = 1024


def _body(l0_ref, l1_ref, feat_full_ref, feat_blk_ref, w1_ref, w2_ref,
          bias_ref, o_ref):
    kh = l0_ref.shape[1]
    x = jnp.dot(l0_ref[...], feat_full_ref[0:kh, :],
                preferred_element_type=jnp.float32)
    x += jnp.dot(l1_ref[...], feat_full_ref[kh:2 * kh, :],
                 preferred_element_type=jnp.float32)
    e = feat_blk_ref[...]
    dn = (((1,), (1,)), ((), ()))
    part1 = jax.lax.dot_general(e + x, w1_ref[...], dn,
                                preferred_element_type=jnp.float32)
    part2 = jax.lax.dot_general(x * e, w2_ref[...], dn,
                                preferred_element_type=jnp.float32)
    o_ref[...] = part1 + part2 + bias_ref[...]


@jax.jit
def kernel(lap_matrix, eye_matrix, features, W1, b1, W2, b2):
    del eye_matrix
    n, d_in = features.shape
    d_out = W1.shape[0]
    bias = (b1 + b2).reshape(1, d_out)
    nh = n // 2

    grid = (n // _BM,)
    return pl.pallas_call(
        _body,
        grid=grid,
        in_specs=[
            pl.BlockSpec((_BM, nh), lambda i: (i, 0)),
            pl.BlockSpec((_BM, nh), lambda i: (i, 1)),
            pl.BlockSpec((n, d_in), lambda i: (0, 0)),
            pl.BlockSpec((_BM, d_in), lambda i: (i, 0)),
            pl.BlockSpec((d_out, d_in), lambda i: (0, 0)),
            pl.BlockSpec((d_out, d_in), lambda i: (0, 0)),
            pl.BlockSpec((1, d_out), lambda i: (0, 0)),
        ],
        out_specs=pl.BlockSpec((_BM, d_out), lambda i: (i, 0)),
        out_shape=jax.ShapeDtypeStruct((n, d_out), jnp.float32),
    )(lap_matrix, lap_matrix, features, features, W1, W2, bias)


# speedup vs baseline: 1.0132x; 1.0132x over previous
import jax
import jax.numpy as jnp
from jax.experimental import pallas as pl

_BM = 512
_NSTREAM = 4


def _body(*refs):
    l_refs = refs[:_NSTREAM]
    feat_full_ref, feat_blk_ref, w1_ref, w2_ref, bias_ref, o_ref = refs[_NSTREAM:]
    kh = l_refs[0].shape[1]
    x = jnp.dot(l_refs[0][...], feat_full_ref[0:kh, :],
                preferred_element_type=jnp.float32)
    for s in range(1, _NSTREAM):
        x += jnp.dot(l_refs[s][...], feat_full_ref[s * kh:(s + 1) * kh, :],
                     preferred_element_type=jnp.float32)
    e = feat_blk_ref[...]
    dn = (((1,), (1,)), ((), ()))
    part1 = jax.lax.dot_general(e + x, w1_ref[...], dn,
                                preferred_element_type=jnp.float32)
    part2 = jax.lax.dot_general(x * e, w2_ref[...], dn,
                                preferred_element_type=jnp.float32)
    o_ref[...] = part1 + part2 + bias_ref[...]


def _make_l_spec(s, n):
    return pl.BlockSpec((_BM, n // _NSTREAM), lambda i, s=s: (i, s))


@jax.jit
def kernel(lap_matrix, eye_matrix, features, W1, b1, W2, b2):
    del eye_matrix
    n, d_in = features.shape
    d_out = W1.shape[0]
    bias = (b1 + b2).reshape(1, d_out)

    grid = (n // _BM,)
    return pl.pallas_call(
        _body,
        grid=grid,
        in_specs=[_make_l_spec(s, n) for s in range(_NSTREAM)] + [
            pl.BlockSpec((n, d_in), lambda i: (0, 0)),
            pl.BlockSpec((_BM, d_in), lambda i: (i, 0)),
            pl.BlockSpec((d_out, d_in), lambda i: (0, 0)),
            pl.BlockSpec((d_out, d_in), lambda i: (0, 0)),
            pl.BlockSpec((1, d_out), lambda i: (0, 0)),
        ],
        out_specs=pl.BlockSpec((_BM, d_out), lambda i: (i, 0)),
        out_shape=jax.ShapeDtypeStruct((n, d_out), jnp.float32),
    )(*([lap_matrix] * _NSTREAM), features, features, W1, W2, bias)


# speedup vs baseline: 1.0162x; 1.0029x over previous
import jax
import jax.numpy as jnp
from jax.experimental import pallas as pl

_BM = 512
_NSTREAM = 4


def _body(*refs):
    l_refs = refs[:_NSTREAM]
    feat_full_ref, w1_ref, w2_ref, bias_ref, o_ref = refs[_NSTREAM:]
    kh = l_refs[0].shape[1]
    x = jnp.dot(l_refs[0][...], feat_full_ref[0:kh, :],
                preferred_element_type=jnp.float32)
    for s in range(1, _NSTREAM):
        x += jnp.dot(l_refs[s][...], feat_full_ref[s * kh:(s + 1) * kh, :],
                     preferred_element_type=jnp.float32)
    i = pl.program_id(0)
    e = feat_full_ref[pl.ds(i * _BM, _BM), :]
    dn = (((1,), (1,)), ((), ()))
    part1 = jax.lax.dot_general(e + x, w1_ref[...], dn,
                                preferred_element_type=jnp.float32)
    part2 = jax.lax.dot_general(x * e, w2_ref[...], dn,
                                preferred_element_type=jnp.float32)
    o_ref[...] = part1 + part2 + bias_ref[...]


def _make_l_spec(s, n):
    return pl.BlockSpec((_BM, n // _NSTREAM), lambda i, s=s: (i, s))


@jax.jit
def kernel(lap_matrix, eye_matrix, features, W1, b1, W2, b2):
    del eye_matrix
    n, d_in = features.shape
    d_out = W1.shape[0]
    bias = (b1 + b2).reshape(1, d_out)

    grid = (n // _BM,)
    return pl.pallas_call(
        _body,
        grid=grid,
        in_specs=[_make_l_spec(s, n) for s in range(_NSTREAM)] + [
            pl.BlockSpec((n, d_in), lambda i: (0, 0)),
            pl.BlockSpec((d_out, d_in), lambda i: (0, 0)),
            pl.BlockSpec((d_out, d_in), lambda i: (0, 0)),
            pl.BlockSpec((1, d_out), lambda i: (0, 0)),
        ],
        out_specs=pl.BlockSpec((_BM, d_out), lambda i: (i, 0)),
        out_shape=jax.ShapeDtypeStruct((n, d_out), jnp.float32),
    )(*([lap_matrix] * _NSTREAM), features, W1, W2, bias)
